# baseline (speedup 1.0000x reference)
import functools

import jax
import jax.numpy as jnp
from jax import lax
from jax.experimental import pallas as pl
from jax.experimental.pallas import tpu as pltpu
from jax.experimental.pallas import tpu_sc as plsc

CROP = 7
LANES = 16
NW = 32
TAPS = 4
CELLS = CROP * CROP
CPAD = 52
TPB = CPAD * TAPS
PART_T = (64, 64, 80)
PART_C = (16, 16, 20)
PART_TO = (0, 64, 128)
PART_CO = (0, 16, 32)
BPW = 32

_DIMNUMS = lax.GatherDimensionNumbers(
    offset_dims=(), collapsed_slice_dims=(0,), start_index_map=(0,))


def _lane_gather(v, idx):
    return lax.gather(v, idx[:, None], _DIMNUMS, (1,),
                      mode=lax.GatherScatterMode.PROMISE_IN_BOUNDS)


def _make_sc_kernel(C, HWs, out_structs):
    NCH = C // LANES
    mesh = plsc.VectorSubcoreMesh(core_axis_name="c", subcore_axis_name="s")

    @functools.partial(
        pl.kernel, mesh=mesh,
        out_type=tuple(out_structs),
        scratch_types=[
            pltpu.VMEM((BPW * LANES,), jnp.float32),
            pltpu.VMEM((BPW * TPB,), jnp.int32),
            pltpu.VMEM((BPW * TPB,), jnp.float32),
            pltpu.VMEM((PART_T[0], C), jnp.float32),
            pltpu.VMEM((PART_T[1], C), jnp.float32),
            pltpu.VMEM((PART_T[2], C), jnp.float32),
            pltpu.VMEM((8, CROP, C), jnp.float32),
            pltpu.VMEM((8, CROP, C), jnp.float32),
            pltpu.SemaphoreType.DMA,
            pltpu.SemaphoreType.DMA,
            pltpu.SemaphoreType.DMA,
            pltpu.SemaphoreType.DMA,
            pltpu.SemaphoreType.DMA,
        ],
    )
    def sc_kernel(t2, t3, t4, t5, boxsc,
                  o2, o3, o4, o5, box_v, idx_v, w_v, g0, g1, g2, ob0, ob1,
                  sg0, sg1, sg2, so0, so1):
        wid = lax.axis_index("s") * 2 + lax.axis_index("c")
        iota = lax.iota(jnp.int32, LANES)
        tvec = (iota.astype(jnp.float32) + 0.5) * (1.0 / CROP)
        uvec = 1.0 - tvec

        for li, (tab, out) in enumerate(((t2, o2), (t3, o3), (t4, o4), (t5, o5))):
            Hl, Wl = HWs[li]
            bbase = (wid // 16) * (Hl * Wl)
            pltpu.sync_copy(boxsc.at[li, wid], box_v)

            def box_meta(n):
                bv = box_v[pl.ds(LANES * n, LANES)]
                y1s, x1s, y2s, x2s = bv[0], bv[1], bv[2], bv[3]
                gy = uvec * y1s + tvec * y2s
                gx = uvec * x1s + tvec * x2s
                yi = gy.astype(jnp.int32)
                y0 = jnp.minimum(yi, Hl - 1)
                y1i = jnp.minimum(yi + 1, Hl - 1)
                xi = gx.astype(jnp.int32)
                x0 = jnp.minimum(xi, Wl - 1)
                x1i = jnp.minimum(xi + 1, Wl - 1)
                ly = gy - y0.astype(jnp.float32)
                hy = 1.0 - ly
                lx = gx - x0.astype(jnp.float32)
                hx = 1.0 - lx
                ry0 = y0 * Wl + bbase
                ry1 = y1i * Wl + bbase
                for k in range(TPB // LANES):
                    slot = iota + LANES * k
                    cell = slot >> 2
                    i = (cell * 9363) >> 16
                    j = cell - i * 7
                    p0 = (slot & 2) == 0
                    q0 = (slot & 1) == 0
                    ri = jnp.where(p0, _lane_gather(ry0, i), _lane_gather(ry1, i))
                    rx = jnp.where(q0, _lane_gather(x0, j), _lane_gather(x1i, j))
                    idx_v[pl.ds(n * TPB + LANES * k, LANES)] = ri + rx
                    wy = jnp.where(p0, _lane_gather(hy, i), _lane_gather(ly, i))
                    wx = jnp.where(q0, _lane_gather(hx, j), _lane_gather(lx, j))
                    w_v[pl.ds(n * TPB + LANES * k, LANES)] = wy * wx

            GBUFS = (g0, g1, g2)
            GSEMS = (sg0, sg1, sg2)

            def gissue(n, part, tab=tab):
                off = n * TPB + PART_TO[part]
                pltpu.async_copy(
                    tab.at[idx_v.at[pl.ds(off, PART_T[part])]],
                    GBUFS[part], GSEMS[part])

            def gwait(n, part, tab=tab):
                off = n * TPB + PART_TO[part]
                pltpu.make_async_copy(
                    tab.at[idx_v.at[pl.ds(off, PART_T[part])]],
                    GBUFS[part], GSEMS[part]).wait()

            def compute_part(n, part, obuf):
                gbuf = GBUFS[part]
                coff = PART_CO[part]
                woff = n * TPB + PART_TO[part]

                @plsc.parallel_loop(0, PART_C[part] // 4)
                def group(m):
                    wv = w_v[pl.ds(woff + LANES * m, LANES)]
                    ws = [(wv[4 * cc], wv[4 * cc + 1], wv[4 * cc + 2],
                           wv[4 * cc + 3]) for cc in range(4)]

                    @plsc.parallel_loop(0, NCH, unroll=2)
                    def chunk(k):
                        sl = pl.ds(LANES * k, LANES)
                        for cc in range(4):
                            row = LANES * m + 4 * cc
                            cell = coff + 4 * m + cc
                            ci = (cell * 9363) >> 16
                            cj = cell - ci * 7
                            w0, w1, w2, w3 = ws[cc]
                            acc = (w0 * gbuf[row, sl] + w1 * gbuf[row + 1, sl]) \
                                + (w2 * gbuf[row + 2, sl] + w3 * gbuf[row + 3, sl])
                            obuf[ci, cj, sl] = acc

            def owait(obuf, sem, out=out):
                pltpu.make_async_copy(
                    obuf.at[pl.ds(0, CROP)], out.at[0], sem).wait()

            lax.fori_loop(0, 2, lambda nn, _: (box_meta(nn), 0)[1], 0)
            for part in range(3):
                gissue(0, part)

            def pair(h, _, tab=tab, out=out):
                for par, ob, so in ((0, ob0, so0), (1, ob1, so1)):
                    n = 2 * h + par

                    @pl.when(h > 0)
                    def _(ob=ob, so=so):
                        owait(ob, so)

                    @pl.when(n + 2 < BPW)
                    def _(n=n):
                        box_meta(n + 2)

                    for part in range(3):
                        gwait(n, part)
                        compute_part(n, part, ob)
                        if par == 0:
                            gissue(n + 1, part)
                        else:
                            @pl.when(n + 1 < BPW)
                            def _(n=n, part=part):
                                gissue(n + 1, part)
                    pltpu.async_copy(
                        ob.at[pl.ds(0, CROP)], out.at[wid * BPW + n], so)
                return 0

            lax.fori_loop(0, BPW // 2, pair, 0)
            owait(ob0, so0)
            owait(ob1, so1)

    return sc_kernel


def kernel(feat_p2, feat_p3, feat_p4, feat_p5, boxes):
    feats = (feat_p2, feat_p3, feat_p4, feat_p5)
    B, N = boxes.shape[:2]
    C = feat_p2.shape[-1]
    M = B * N * CELLS
    assert B * N == NW * BPW
    flat_boxes = boxes.reshape(B * N, TAPS)
    scaled = []
    for lvl in range(2, 6):
        s = flat_boxes * jnp.float32(1.0 / 2.0 ** lvl)
        s = jnp.pad(s, ((0, 0), (0, LANES - TAPS)))
        scaled.append(s.reshape(NW, BPW * LANES))
    boxsc = jnp.stack(scaled)
    tabs = [f.reshape(-1, C) for f in feats]
    HWs = [(f.shape[1], f.shape[2]) for f in feats]
    out_structs = [jax.ShapeDtypeStruct((B * N, CROP, CROP, C), jnp.float32)] * 4
    sc = _make_sc_kernel(C, HWs, out_structs)
    outs = sc(*tabs, boxsc)
    return tuple(o.reshape(B, N, CROP, CROP, C) for o in outs)


# speedup vs baseline: 1.0013x; 1.0013x over previous
import functools

import jax
import jax.numpy as jnp
from jax import lax
from jax.experimental import pallas as pl
from jax.experimental.pallas import tpu as pltpu
from jax.experimental.pallas import tpu_sc as plsc

CROP = 7
LANES = 16
NW = 32
TAPS = 4
CELLS = CROP * CROP
CPAD = 52
TPB = CPAD * TAPS
PART_T = (64, 64, 80)
PART_C = (16, 16, 20)
PART_TO = (0, 64, 128)
PART_CO = (0, 16, 32)
BPW = 32

_DIMNUMS = lax.GatherDimensionNumbers(
    offset_dims=(), collapsed_slice_dims=(0,), start_index_map=(0,))


def _lane_gather(v, idx):
    return lax.gather(v, idx[:, None], _DIMNUMS, (1,),
                      mode=lax.GatherScatterMode.PROMISE_IN_BOUNDS)


def _make_sc_kernel(C, HWs, out_structs):
    NCH = C // LANES
    mesh = plsc.VectorSubcoreMesh(core_axis_name="c", subcore_axis_name="s")

    @functools.partial(
        pl.kernel, mesh=mesh,
        out_type=tuple(out_structs),
        scratch_types=[
            pltpu.VMEM((BPW * LANES,), jnp.float32),
            pltpu.VMEM((BPW * TPB,), jnp.int32),
            pltpu.VMEM((BPW * TPB,), jnp.float32),
            pltpu.VMEM((PART_T[0], C), jnp.float32),
            pltpu.VMEM((PART_T[1], C), jnp.float32),
            pltpu.VMEM((PART_T[2], C), jnp.float32),
            pltpu.VMEM((8, CROP, C), jnp.float32),
            pltpu.VMEM((8, CROP, C), jnp.float32),
            pltpu.SemaphoreType.DMA,
            pltpu.SemaphoreType.DMA,
            pltpu.SemaphoreType.DMA,
            pltpu.SemaphoreType.DMA,
            pltpu.SemaphoreType.DMA,
        ],
    )
    def sc_kernel(t2, t3, t4, t5, boxsc,
                  o2, o3, o4, o5, box_v, idx_v, w_v, g0, g1, g2, ob0, ob1,
                  sg0, sg1, sg2, so0, so1):
        wid = lax.axis_index("s") * 2 + lax.axis_index("c")
        iota = lax.iota(jnp.int32, LANES)
        tvec = (iota.astype(jnp.float32) + 0.5) * (1.0 / CROP)
        uvec = 1.0 - tvec

        for li, (tab, out) in enumerate(((t2, o2), (t3, o3), (t4, o4), (t5, o5))):
            Hl, Wl = HWs[li]
            bbase = (wid // 16) * (Hl * Wl)
            pltpu.sync_copy(boxsc.at[li, wid], box_v)

            def box_meta(n):
                bv = box_v[pl.ds(LANES * n, LANES)]
                y1s, x1s, y2s, x2s = bv[0], bv[1], bv[2], bv[3]
                gy = uvec * y1s + tvec * y2s
                gx = uvec * x1s + tvec * x2s
                yi = gy.astype(jnp.int32)
                y0 = jnp.minimum(yi, Hl - 1)
                y1i = jnp.minimum(yi + 1, Hl - 1)
                xi = gx.astype(jnp.int32)
                x0 = jnp.minimum(xi, Wl - 1)
                x1i = jnp.minimum(xi + 1, Wl - 1)
                ly = gy - y0.astype(jnp.float32)
                hy = 1.0 - ly
                lx = gx - x0.astype(jnp.float32)
                hx = 1.0 - lx
                ry0 = y0 * Wl + bbase
                ry1 = y1i * Wl + bbase
                for k in range(TPB // LANES):
                    slot = iota + LANES * k
                    cell = slot >> 2
                    i = (cell * 9363) >> 16
                    j = cell - i * 7
                    p0 = (slot & 2) == 0
                    q0 = (slot & 1) == 0
                    ri = jnp.where(p0, _lane_gather(ry0, i), _lane_gather(ry1, i))
                    rx = jnp.where(q0, _lane_gather(x0, j), _lane_gather(x1i, j))
                    idx_v[pl.ds(n * TPB + LANES * k, LANES)] = ri + rx
                    wy = jnp.where(p0, _lane_gather(hy, i), _lane_gather(ly, i))
                    wx = jnp.where(q0, _lane_gather(hx, j), _lane_gather(lx, j))
                    w_v[pl.ds(n * TPB + LANES * k, LANES)] = wy * wx

            GBUFS = (g0, g1, g2)
            GSEMS = (sg0, sg1, sg2)

            def gissue(n, part, tab=tab):
                off = n * TPB + PART_TO[part]
                pltpu.async_copy(
                    tab.at[idx_v.at[pl.ds(off, PART_T[part])]],
                    GBUFS[part], GSEMS[part])

            def gwait(n, part, tab=tab):
                off = n * TPB + PART_TO[part]
                pltpu.make_async_copy(
                    tab.at[idx_v.at[pl.ds(off, PART_T[part])]],
                    GBUFS[part], GSEMS[part]).wait()

            def compute_part(n, part, obuf):
                gbuf = GBUFS[part]
                coff = PART_CO[part]
                woff = n * TPB + PART_TO[part]

                @plsc.parallel_loop(0, PART_C[part] // 4)
                def group(m):
                    wv = w_v[pl.ds(woff + LANES * m, LANES)]
                    ws = [(wv[4 * cc], wv[4 * cc + 1], wv[4 * cc + 2],
                           wv[4 * cc + 3]) for cc in range(4)]
                    cij = []
                    for cc in range(4):
                        cell = coff + 4 * m + cc
                        ci = (cell * 9363) >> 16
                        cij.append((ci, cell - ci * 7))

                    @plsc.parallel_loop(0, NCH, unroll=2)
                    def chunk(k):
                        sl = pl.ds(LANES * k, LANES)
                        for cc in range(4):
                            row = LANES * m + 4 * cc
                            ci, cj = cij[cc]
                            w0, w1, w2, w3 = ws[cc]
                            acc = (w0 * gbuf[row, sl] + w1 * gbuf[row + 1, sl]) \
                                + (w2 * gbuf[row + 2, sl] + w3 * gbuf[row + 3, sl])
                            obuf[ci, cj, sl] = acc

            def owait(obuf, sem, out=out):
                pltpu.make_async_copy(
                    obuf.at[pl.ds(0, CROP)], out.at[0], sem).wait()

            lax.fori_loop(0, 2, lambda nn, _: (box_meta(nn), 0)[1], 0)
            for part in range(3):
                gissue(0, part)

            def pair(h, _, tab=tab, out=out):
                for par, ob, so in ((0, ob0, so0), (1, ob1, so1)):
                    n = 2 * h + par

                    @pl.when(h > 0)
                    def _(ob=ob, so=so):
                        owait(ob, so)

                    @pl.when(n + 2 < BPW)
                    def _(n=n):
                        box_meta(n + 2)

                    for part in range(3):
                        gwait(n, part)
                        compute_part(n, part, ob)
                        if par == 0:
                            gissue(n + 1, part)
                        else:
                            @pl.when(n + 1 < BPW)
                            def _(n=n, part=part):
                                gissue(n + 1, part)
                    pltpu.async_copy(
                        ob.at[pl.ds(0, CROP)], out.at[wid * BPW + n], so)
                return 0

            lax.fori_loop(0, BPW // 2, pair, 0)
            owait(ob0, so0)
            owait(ob1, so1)

    return sc_kernel


def kernel(feat_p2, feat_p3, feat_p4, feat_p5, boxes):
    feats = (feat_p2, feat_p3, feat_p4, feat_p5)
    B, N = boxes.shape[:2]
    C = feat_p2.shape[-1]
    M = B * N * CELLS
    assert B * N == NW * BPW
    flat_boxes = boxes.reshape(B * N, TAPS)
    scaled = []
    for lvl in range(2, 6):
        s = flat_boxes * jnp.float32(1.0 / 2.0 ** lvl)
        s = jnp.pad(s, ((0, 0), (0, LANES - TAPS)))
        scaled.append(s.reshape(NW, BPW * LANES))
    boxsc = jnp.stack(scaled)
    tabs = [f.reshape(-1, C) for f in feats]
    HWs = [(f.shape[1], f.shape[2]) for f in feats]
    out_structs = [jax.ShapeDtypeStruct((B * N, CROP, CROP, C), jnp.float32)] * 4
    sc = _make_sc_kernel(C, HWs, out_structs)
    outs = sc(*tabs, boxsc)
    return tuple(o.reshape(B, N, CROP, CROP, C) for o in outs)
